# baseline (speedup 1.0000x reference)
import numpy as np
import jax
import jax.numpy as jnp
from jax.experimental import pallas as pl

OUT_F = 64
HID = 16
TRI = OUT_F * (OUT_F + 1) // 2
TRI_PAD = ((TRI + 127) // 128) * 128
THRESH = 0.005

_tri_rows, _tri_cols = np.tril_indices(OUT_F)
_t = np.arange(TRI)

_COL_OH = np.zeros((OUT_F, TRI_PAD), np.float32)
_COL_OH[_tri_cols, _t] = 1.0
_ROW_OH = np.zeros((TRI_PAD, OUT_F), np.float32)
_ROW_OH[_t, _tri_rows] = 1.0
_FLOOR = np.zeros((1, TRI_PAD), np.float32)
_FLOOR[0, _t[_tri_rows == _tri_cols]] = THRESH


def _feat_kernel(x_ref, w_ref, o_ref):
    o_ref[...] = jnp.dot(x_ref[...], w_ref[...],
                         preferred_element_type=jnp.float32)


def _pass1_kernel(adj_ref, ab_ref, o_ref, adjb_ref):
    a = adj_ref[...]
    acc = jnp.dot(a, ab_ref[...], preferred_element_type=jnp.float32)
    o_ref[...] = jnp.maximum(acc, 0.0)
    adjb_ref[...] = a.astype(jnp.bfloat16)


def _pass2_kernel(adj_ref, h_ref, w2_ref, eps_ref, mean_ref,
                  col_ref, row_ref, floor_ref, o_ref):
    m = jnp.dot(adj_ref[...], h_ref[...],
                preferred_element_type=jnp.float32)
    g = jnp.dot(m, w2_ref[...], preferred_element_type=jnp.float32)
    p = jnp.maximum(g, floor_ref[...])
    e = jnp.dot(eps_ref[...], col_ref[...],
                preferred_element_type=jnp.float32)
    tr = jnp.dot(p * e, row_ref[...],
                 preferred_element_type=jnp.float32)
    o_ref[...] = tr + mean_ref[...]


def kernel(input, adj, W_mean, W1, W2):
    n = adj.shape[0]
    wcat = jnp.concatenate([W_mean, W1], axis=1)
    c = wcat.shape[1]
    w2p = jnp.pad(W2, ((0, 0), (0, TRI_PAD - TRI)))
    eps = jax.random.normal(jax.random.key(1234), (n, OUT_F),
                            dtype=jnp.float32)
    col_oh = jnp.asarray(_COL_OH)
    row_oh = jnp.asarray(_ROW_OH)
    floor = jnp.asarray(_FLOOR)

    ab = pl.pallas_call(
        _feat_kernel,
        out_shape=jax.ShapeDtypeStruct((n, c), jnp.float32),
    )(input, wcat)

    r = 400 if n % 400 == 0 else n
    grid = (n // r,)

    mh, adjb = pl.pallas_call(
        _pass1_kernel,
        grid=grid,
        in_specs=[pl.BlockSpec((r, n), lambda i: (i, 0)),
                  pl.BlockSpec((n, c), lambda i: (0, 0))],
        out_specs=[pl.BlockSpec((r, c), lambda i: (i, 0)),
                   pl.BlockSpec((r, n), lambda i: (i, 0))],
        out_shape=[jax.ShapeDtypeStruct((n, c), jnp.float32),
                   jax.ShapeDtypeStruct((n, n), jnp.bfloat16)],
    )(adj, ab)

    mean = mh[:, :OUT_F]
    h = mh[:, OUT_F:].astype(jnp.bfloat16)

    out = pl.pallas_call(
        _pass2_kernel,
        grid=grid,
        in_specs=[pl.BlockSpec((r, n), lambda i: (i, 0)),
                  pl.BlockSpec((n, HID), lambda i: (0, 0)),
                  pl.BlockSpec((HID, TRI_PAD), lambda i: (0, 0)),
                  pl.BlockSpec((r, OUT_F), lambda i: (i, 0)),
                  pl.BlockSpec((r, OUT_F), lambda i: (i, 0)),
                  pl.BlockSpec((OUT_F, TRI_PAD), lambda i: (0, 0)),
                  pl.BlockSpec((TRI_PAD, OUT_F), lambda i: (0, 0)),
                  pl.BlockSpec((1, TRI_PAD), lambda i: (0, 0))],
        out_specs=pl.BlockSpec((r, OUT_F), lambda i: (i, 0)),
        out_shape=jax.ShapeDtypeStruct((n, OUT_F), jnp.float32),
    )(adjb, h, w2p, eps, mean, col_oh, row_oh, floor)
    return out


# speedup vs baseline: 1.1538x; 1.1538x over previous
import numpy as np
import jax
import jax.numpy as jnp
from jax.experimental import pallas as pl

OUT_F = 64
HID = 16
TRI = OUT_F * (OUT_F + 1) // 2
TRI_PAD = ((TRI + 127) // 128) * 128
THRESH = 0.005
MEAN_LO = 32

_tri_rows, _tri_cols = np.tril_indices(OUT_F)
_t = np.arange(TRI)

_COL_OH = np.zeros((OUT_F, TRI_PAD), np.float32)
_COL_OH[_tri_cols, _t] = 1.0
_ROW_OH = np.zeros((TRI_PAD, OUT_F), np.float32)
_ROW_OH[_t, _tri_rows] = 1.0
_FLOOR = np.zeros((1, TRI_PAD), np.float32)
_FLOOR[0, _t[_tri_rows == _tri_cols]] = THRESH


def _feat_kernel(x_ref, w_ref, o_ref):
    o_ref[...] = jnp.dot(x_ref[...], w_ref[...],
                         preferred_element_type=jnp.float32)


def _pass1_kernel(adj_ref, ab_ref, o_ref):
    acc = jnp.dot(adj_ref[...], ab_ref[...],
                  preferred_element_type=jnp.float32)
    o_ref[...] = jnp.maximum(acc, 0.0)


def _pass2_kernel(adj_ref, ah_ref, w2_ref, eps_ref, mlo_ref,
                  col_ref, row_ref, floor_ref, o_ref):
    ma = jnp.dot(adj_ref[...], ah_ref[...],
                 preferred_element_type=jnp.float32)
    m = ma[:, :HID]
    mean_hi = jnp.maximum(ma[:, HID:], 0.0)
    g = jnp.dot(m, w2_ref[...], preferred_element_type=jnp.float32)
    p = jnp.maximum(g, floor_ref[...])
    e = jnp.dot(eps_ref[...], col_ref[...],
                preferred_element_type=jnp.float32)
    tr = jnp.dot(p * e, row_ref[...],
                 preferred_element_type=jnp.float32)
    mean = jnp.concatenate([mlo_ref[...], mean_hi], axis=1)
    o_ref[...] = tr + mean


def kernel(input, adj, W_mean, W1, W2):
    n = adj.shape[0]
    wcat = jnp.concatenate([W1, W_mean], axis=1)
    c = wcat.shape[1]
    c1 = HID + MEAN_LO
    w2p = jnp.pad(W2, ((0, 0), (0, TRI_PAD - TRI)))
    eps = jax.random.normal(jax.random.key(1234), (n, OUT_F),
                            dtype=jnp.float32)
    col_oh = jnp.asarray(_COL_OH)
    row_oh = jnp.asarray(_ROW_OH)
    floor = jnp.asarray(_FLOOR)

    ab = pl.pallas_call(
        _feat_kernel,
        out_shape=jax.ShapeDtypeStruct((n, c), jnp.float32),
    )(input, wcat)

    r = 400 if n % 400 == 0 else n
    grid = (n // r,)

    hm = pl.pallas_call(
        _pass1_kernel,
        grid=grid,
        in_specs=[pl.BlockSpec((r, n), lambda i: (i, 0)),
                  pl.BlockSpec((n, c1), lambda i: (0, 0))],
        out_specs=pl.BlockSpec((r, c1), lambda i: (i, 0)),
        out_shape=jax.ShapeDtypeStruct((n, c1), jnp.float32),
    )(adj, ab[:, :c1])

    h = hm[:, :HID]
    mean_lo = hm[:, HID:]
    ah = jnp.concatenate([h, ab[:, c1:]], axis=1)

    out = pl.pallas_call(
        _pass2_kernel,
        grid=grid,
        in_specs=[pl.BlockSpec((r, n), lambda i: (i, 0)),
                  pl.BlockSpec((n, c1), lambda i: (0, 0)),
                  pl.BlockSpec((HID, TRI_PAD), lambda i: (0, 0)),
                  pl.BlockSpec((r, OUT_F), lambda i: (i, 0)),
                  pl.BlockSpec((r, MEAN_LO), lambda i: (i, 0)),
                  pl.BlockSpec((OUT_F, TRI_PAD), lambda i: (0, 0)),
                  pl.BlockSpec((TRI_PAD, OUT_F), lambda i: (0, 0)),
                  pl.BlockSpec((1, TRI_PAD), lambda i: (0, 0))],
        out_specs=pl.BlockSpec((r, OUT_F), lambda i: (i, 0)),
        out_shape=jax.ShapeDtypeStruct((n, OUT_F), jnp.float32),
    )(adj, ah, w2p, eps, mean_lo, col_oh, row_oh, floor)
    return out


# speedup vs baseline: 1.1962x; 1.0367x over previous
import numpy as np
import jax
import jax.numpy as jnp
from jax.experimental import pallas as pl
from jax.experimental.pallas import tpu as pltpu

OUT_F = 64
HID = 16
TRI = OUT_F * (OUT_F + 1) // 2
TRI_PAD = ((TRI + 127) // 128) * 128
THRESH = 0.005

_tri_rows, _tri_cols = np.tril_indices(OUT_F)
_t = np.arange(TRI)

_COL_OH = np.zeros((OUT_F, TRI_PAD), np.float32)
_COL_OH[_tri_cols, _t] = 1.0
_ROW_OH = np.zeros((TRI_PAD, OUT_F), np.float32)
_ROW_OH[_t, _tri_rows] = 1.0
_FLOOR = np.zeros((1, TRI_PAD), np.float32)
_FLOOR[0, _t[_tri_rows == _tri_cols]] = THRESH


def _feat_kernel(x_ref, w_ref, o_ref):
    o_ref[...] = jnp.dot(x_ref[...], w_ref[...],
                         preferred_element_type=jnp.float32)


def _make_fused_kernel(r):
    def _fused_kernel(adj_ref, ab_ref, w2_ref, eps_ref,
                      col_ref, row_ref, floor_ref, o_ref,
                      mean_scr, h_scr):
        p = pl.program_id(0)
        i = pl.program_id(1)

        @pl.when(p == 0)
        def _():
            mh = jnp.maximum(
                jnp.dot(adj_ref[...], ab_ref[...],
                        preferred_element_type=jnp.float32), 0.0)
            mean_scr[pl.ds(i * r, r), :] = mh[:, :OUT_F]
            h_scr[pl.ds(i * r, r), :] = mh[:, OUT_F:]

        @pl.when(p == 1)
        def _():
            m = jnp.dot(adj_ref[...], h_scr[...],
                        preferred_element_type=jnp.float32)
            g = jnp.dot(m, w2_ref[...],
                        preferred_element_type=jnp.float32)
            pmat = jnp.maximum(g, floor_ref[...])
            e = jnp.dot(eps_ref[...], col_ref[...],
                        preferred_element_type=jnp.float32)
            tr = jnp.dot(pmat * e, row_ref[...],
                         preferred_element_type=jnp.float32)
            o_ref[...] = tr + mean_scr[pl.ds(i * r, r), :]

    return _fused_kernel


def kernel(input, adj, W_mean, W1, W2):
    n = adj.shape[0]
    wcat = jnp.concatenate([W_mean, W1], axis=1)
    c = wcat.shape[1]
    w2p = jnp.pad(W2, ((0, 0), (0, TRI_PAD - TRI)))
    eps = jax.random.normal(jax.random.key(1234), (n, OUT_F),
                            dtype=jnp.float32)
    col_oh = jnp.asarray(_COL_OH)
    row_oh = jnp.asarray(_ROW_OH)
    floor = jnp.asarray(_FLOOR)

    ab = pl.pallas_call(
        _feat_kernel,
        out_shape=jax.ShapeDtypeStruct((n, c), jnp.float32),
    )(input, wcat)

    r = 400 if n % 400 == 0 else n
    grid = (2, n // r)

    out = pl.pallas_call(
        _make_fused_kernel(r),
        grid=grid,
        in_specs=[pl.BlockSpec((r, n), lambda p, i: (i, 0)),
                  pl.BlockSpec((n, c), lambda p, i: (0, 0)),
                  pl.BlockSpec((HID, TRI_PAD), lambda p, i: (0, 0)),
                  pl.BlockSpec((r, OUT_F), lambda p, i: (i, 0)),
                  pl.BlockSpec((OUT_F, TRI_PAD), lambda p, i: (0, 0)),
                  pl.BlockSpec((TRI_PAD, OUT_F), lambda p, i: (0, 0)),
                  pl.BlockSpec((1, TRI_PAD), lambda p, i: (0, 0))],
        out_specs=pl.BlockSpec((r, OUT_F), lambda p, i: (i, 0)),
        out_shape=jax.ShapeDtypeStruct((n, OUT_F), jnp.float32),
        scratch_shapes=[pltpu.VMEM((n, OUT_F), jnp.float32),
                        pltpu.VMEM((n, HID), jnp.float32)],
    )(adj, ab, w2p, eps, col_oh, row_oh, floor)
    return out


# speedup vs baseline: 1.2098x; 1.0114x over previous
import numpy as np
import jax
import jax.numpy as jnp
from jax.experimental import pallas as pl
from jax.experimental.pallas import tpu as pltpu

OUT_F = 64
HID = 16
TRI = OUT_F * (OUT_F + 1) // 2
TRI_PAD = ((TRI + 127) // 128) * 128
THRESH = 0.005

_tri_rows, _tri_cols = np.tril_indices(OUT_F)
_t = np.arange(TRI)

_COL_OH = np.zeros((OUT_F, TRI_PAD), np.float32)
_COL_OH[_tri_cols, _t] = 1.0
_ROW_OH = np.zeros((TRI_PAD, OUT_F), np.float32)
_ROW_OH[_t, _tri_rows] = 1.0
_FLOOR = np.zeros((1, TRI_PAD), np.float32)
_FLOOR[0, _t[_tri_rows == _tri_cols]] = THRESH


def _make_fused_kernel(r):
    def _fused_kernel(adj_ref, x_ref, w_ref, w2_ref, eps_ref,
                      col_ref, row_ref, floor_ref, o_ref,
                      mean_scr, h_scr, ab_scr):
        p = pl.program_id(0)
        i = pl.program_id(1)

        @pl.when((p == 0) & (i == 0))
        def _():
            ab_scr[...] = jnp.dot(x_ref[...], w_ref[...],
                                  preferred_element_type=jnp.float32)

        @pl.when(p == 0)
        def _():
            mh = jnp.maximum(
                jnp.dot(adj_ref[...], ab_scr[...],
                        preferred_element_type=jnp.float32), 0.0)
            mean_scr[pl.ds(i * r, r), :] = mh[:, :OUT_F]
            h_scr[pl.ds(i * r, r), :] = mh[:, OUT_F:]

        @pl.when(p == 1)
        def _():
            m = jnp.dot(adj_ref[...], h_scr[...],
                        preferred_element_type=jnp.float32)
            g = jnp.dot(m, w2_ref[...],
                        preferred_element_type=jnp.float32)
            pmat = jnp.maximum(g, floor_ref[...])
            e = jnp.dot(eps_ref[...], col_ref[...],
                        preferred_element_type=jnp.float32)
            tr = jnp.dot(pmat * e, row_ref[...],
                         preferred_element_type=jnp.float32)
            o_ref[...] = tr + mean_scr[pl.ds(i * r, r), :]

    return _fused_kernel


def kernel(input, adj, W_mean, W1, W2):
    n = adj.shape[0]
    wcat = jnp.concatenate([W_mean, W1], axis=1)
    c = wcat.shape[1]
    w2p = jnp.pad(W2, ((0, 0), (0, TRI_PAD - TRI)))
    eps = jax.random.normal(jax.random.key(1234), (n, OUT_F),
                            dtype=jnp.float32)
    col_oh = jnp.asarray(_COL_OH)
    row_oh = jnp.asarray(_ROW_OH)
    floor = jnp.asarray(_FLOOR)

    r = 400 if n % 400 == 0 else n
    grid = (2, n // r)

    out = pl.pallas_call(
        _make_fused_kernel(r),
        grid=grid,
        in_specs=[pl.BlockSpec((r, n), lambda p, i: (i, 0)),
                  pl.BlockSpec((n, input.shape[1]), lambda p, i: (0, 0)),
                  pl.BlockSpec((input.shape[1], c), lambda p, i: (0, 0)),
                  pl.BlockSpec((HID, TRI_PAD), lambda p, i: (0, 0)),
                  pl.BlockSpec((r, OUT_F), lambda p, i: (p * i, 0)),
                  pl.BlockSpec((OUT_F, TRI_PAD), lambda p, i: (0, 0)),
                  pl.BlockSpec((TRI_PAD, OUT_F), lambda p, i: (0, 0)),
                  pl.BlockSpec((1, TRI_PAD), lambda p, i: (0, 0))],
        out_specs=pl.BlockSpec((r, OUT_F), lambda p, i: (p * i, 0)),
        out_shape=jax.ShapeDtypeStruct((n, OUT_F), jnp.float32),
        scratch_shapes=[pltpu.VMEM((n, OUT_F), jnp.float32),
                        pltpu.VMEM((n, HID), jnp.float32),
                        pltpu.VMEM((n, c), jnp.float32)],
    )(adj, input, wcat, w2p, eps, col_oh, row_oh, floor)
    return out
